# baseline (speedup 1.0000x reference)
import functools

import jax
import jax.numpy as jnp
from jax import lax
from jax.experimental import pallas as pl
from jax.experimental.pallas import tpu as pltpu
from jax.experimental.pallas import tpu_sc as plsc

N_TOKENS = 4608
N_CODES = 8192
D = 256
R = 384
NB = N_TOKENS // R
BETA = 0.25

NC = 2
NS = 16
NW = NC * NS
B_PER_W = N_TOKENS // NW


def _vq_body(zf_ref, w_ref, idx_ref, enc_ref, loss_ref, perp_ref,
             w_s, wn_s, cnt_s, dacc_s, w_sem):
    i = pl.program_id(0)
    zb = zf_ref[...]

    @pl.when(i == 0)
    def _():
        pltpu.make_async_copy(w_ref, w_s, w_sem).start()
        pltpu.make_async_copy(w_ref, w_s, w_sem).wait()
        w0 = w_s[...]
        wn_s[...] = jnp.reshape(jnp.sum(w0 * w0, axis=1), (1, N_CODES))
        cnt_s[...] = jnp.zeros_like(cnt_s)
        dacc_s[...] = jnp.zeros_like(dacc_s)

    w = w_s[...]

    znorm = jnp.sum(zb * zb, axis=1, keepdims=True)
    m2 = lax.dot_general(zb * -2.0, w, (((1,), (1,)), ((), ())),
                         preferred_element_type=jnp.float32)
    d = (znorm + wn_s[...]) + m2
    dmin = jnp.min(d, axis=1, keepdims=True)
    iota = lax.broadcasted_iota(jnp.int32, (R, N_CODES), 1)
    idx = jnp.min(jnp.where(d == dmin, iota, N_CODES), axis=1)
    idx_ref[0, 0, :] = idx
    onehot = (iota == idx[:, None]).astype(jnp.float32)
    enc_ref[...] = onehot
    ones_r = jnp.full((1, R), 1.0, jnp.float32)
    cnt_s[...] += lax.dot_general(ones_r, onehot, (((1,), (0,)), ((), ())),
                                  preferred_element_type=jnp.float32)
    dacc_s[...] += dmin

    @pl.when(i == NB - 1)
    def _():
        loss_ref[...] = jnp.reshape(
            jnp.sum(dacc_s[...]) * ((1.0 + BETA) / (N_TOKENS * D)), (1, 1))
        p = cnt_s[...] * (1.0 / N_TOKENS)
        perp_ref[...] = jnp.reshape(
            jnp.exp(-jnp.sum(p * jnp.log(p + 1e-10))), (1, 1))


def _vq_tc(zf, wt):
    return pl.pallas_call(
        _vq_body,
        grid=(NB,),
        in_specs=[
            pl.BlockSpec((R, D), lambda i: (i, 0)),
            pl.BlockSpec(memory_space=pl.ANY),
        ],
        out_specs=[
            pl.BlockSpec((1, 1, R), lambda i: (i, 0, 0)),
            pl.BlockSpec((R, N_CODES), lambda i: (i, 0)),
            pl.BlockSpec((1, 1), lambda i: (0, 0)),
            pl.BlockSpec((1, 1), lambda i: (0, 0)),
        ],
        out_shape=[
            jax.ShapeDtypeStruct((NB, 1, R), jnp.int32),
            jax.ShapeDtypeStruct((N_TOKENS, N_CODES), jnp.float32),
            jax.ShapeDtypeStruct((1, 1), jnp.float32),
            jax.ShapeDtypeStruct((1, 1), jnp.float32),
        ],
        scratch_shapes=[
            pltpu.VMEM((N_CODES, D), jnp.float32),
            pltpu.VMEM((1, N_CODES), jnp.float32),
            pltpu.VMEM((1, N_CODES), jnp.float32),
            pltpu.VMEM((R, 1), jnp.float32),
            pltpu.SemaphoreType.DMA,
        ],
    )(zf, wt)


def _sc_gather_body(table_hbm, idx_hbm, out_hbm, idx_v, rows_v, sem):
    wid = lax.axis_index("s") * NC + lax.axis_index("c")
    base = wid * B_PER_W
    pltpu.sync_copy(idx_hbm.at[pl.ds(base, B_PER_W)], idx_v)
    pltpu.async_copy(table_hbm.at[idx_v], rows_v, sem).wait()
    pltpu.sync_copy(rows_v, out_hbm.at[pl.ds(base, B_PER_W)])


@functools.cache
def _sc_gather():
    return pl.kernel(
        _sc_gather_body,
        mesh=plsc.VectorSubcoreMesh(core_axis_name="c", subcore_axis_name="s",
                                    num_cores=NC),
        out_type=jax.ShapeDtypeStruct((N_TOKENS, D), jnp.float32),
        scratch_types=[
            pltpu.VMEM((B_PER_W,), jnp.int32),
            pltpu.VMEM((B_PER_W, D), jnp.float32),
            pltpu.SemaphoreType.DMA,
        ],
    )


def kernel(z, weight):
    zf = jnp.transpose(z, (0, 2, 3, 1)).reshape(N_TOKENS, D)
    idx3, enc, loss11, perp11 = _vq_tc(zf, weight)
    idx = idx3.reshape(N_TOKENS)
    z_q = _sc_gather()(weight, idx)
    z_q_out = jnp.transpose(z_q.reshape(8, 24, 24, D), (0, 3, 1, 2))
    return (z_q_out, loss11[0, 0], (perp11[0, 0], enc, idx))


# speedup vs baseline: 1.0045x; 1.0045x over previous
import functools

import jax
import jax.numpy as jnp
from jax import lax
from jax.experimental import pallas as pl
from jax.experimental.pallas import tpu as pltpu
from jax.experimental.pallas import tpu_sc as plsc

N_TOKENS = 4608
N_CODES = 8192
D = 256
R = 256
NB = N_TOKENS // R
BETA = 0.25

NC = 2
NS = 16
NW = NC * NS
B_PER_W = N_TOKENS // NW


def _vq_body(zf_ref, w_ref, idx_ref, enc_ref, loss_ref, perp_ref,
             w_s, wn_s, cnt_s, dacc_s, w_sem):
    i = pl.program_id(0)
    zb = zf_ref[...]

    @pl.when(i == 0)
    def _():
        pltpu.make_async_copy(w_ref, w_s, w_sem).start()
        pltpu.make_async_copy(w_ref, w_s, w_sem).wait()
        w0 = w_s[...]
        wn_s[...] = jnp.reshape(jnp.sum(w0 * w0, axis=1), (1, N_CODES))
        cnt_s[...] = jnp.zeros_like(cnt_s)
        dacc_s[...] = jnp.zeros_like(dacc_s)

    w = w_s[...]

    znorm = jnp.sum(zb * zb, axis=1, keepdims=True)
    m2 = lax.dot_general(zb * -2.0, w, (((1,), (1,)), ((), ())),
                         preferred_element_type=jnp.float32)
    d = (znorm + wn_s[...]) + m2
    dmin = jnp.min(d, axis=1, keepdims=True)
    iota = lax.broadcasted_iota(jnp.int32, (R, N_CODES), 1)
    idx = jnp.min(jnp.where(d == dmin, iota, N_CODES), axis=1)
    idx_ref[0, 0, :] = idx
    onehot = (iota == idx[:, None]).astype(jnp.float32)
    enc_ref[...] = onehot
    ones_r = jnp.full((1, R), 1.0, jnp.float32)
    cnt_s[...] += lax.dot_general(ones_r, onehot, (((1,), (0,)), ((), ())),
                                  preferred_element_type=jnp.float32)
    dacc_s[...] += dmin

    @pl.when(i == NB - 1)
    def _():
        loss_ref[...] = jnp.reshape(
            jnp.sum(dacc_s[...]) * ((1.0 + BETA) / (N_TOKENS * D)), (1, 1))
        p = cnt_s[...] * (1.0 / N_TOKENS)
        perp_ref[...] = jnp.reshape(
            jnp.exp(-jnp.sum(p * jnp.log(p + 1e-10))), (1, 1))


def _vq_tc(zf, wt):
    return pl.pallas_call(
        _vq_body,
        grid=(NB,),
        in_specs=[
            pl.BlockSpec((R, D), lambda i: (i, 0)),
            pl.BlockSpec(memory_space=pl.ANY),
        ],
        out_specs=[
            pl.BlockSpec((1, 1, R), lambda i: (i, 0, 0)),
            pl.BlockSpec((R, N_CODES), lambda i: (i, 0)),
            pl.BlockSpec((1, 1), lambda i: (0, 0)),
            pl.BlockSpec((1, 1), lambda i: (0, 0)),
        ],
        out_shape=[
            jax.ShapeDtypeStruct((NB, 1, R), jnp.int32),
            jax.ShapeDtypeStruct((N_TOKENS, N_CODES), jnp.float32),
            jax.ShapeDtypeStruct((1, 1), jnp.float32),
            jax.ShapeDtypeStruct((1, 1), jnp.float32),
        ],
        scratch_shapes=[
            pltpu.VMEM((N_CODES, D), jnp.float32),
            pltpu.VMEM((1, N_CODES), jnp.float32),
            pltpu.VMEM((1, N_CODES), jnp.float32),
            pltpu.VMEM((R, 1), jnp.float32),
            pltpu.SemaphoreType.DMA,
        ],
    )(zf, wt)


def _sc_gather_body(table_hbm, idx_hbm, out_hbm, idx_v, rows_v, sem):
    wid = lax.axis_index("s") * NC + lax.axis_index("c")
    base = wid * B_PER_W
    pltpu.sync_copy(idx_hbm.at[pl.ds(base, B_PER_W)], idx_v)
    pltpu.async_copy(table_hbm.at[idx_v], rows_v, sem).wait()
    pltpu.sync_copy(rows_v, out_hbm.at[pl.ds(base, B_PER_W)])


@functools.cache
def _sc_gather():
    return pl.kernel(
        _sc_gather_body,
        mesh=plsc.VectorSubcoreMesh(core_axis_name="c", subcore_axis_name="s",
                                    num_cores=NC),
        out_type=jax.ShapeDtypeStruct((N_TOKENS, D), jnp.float32),
        scratch_types=[
            pltpu.VMEM((B_PER_W,), jnp.int32),
            pltpu.VMEM((B_PER_W, D), jnp.float32),
            pltpu.SemaphoreType.DMA,
        ],
    )


def kernel(z, weight):
    zf = jnp.transpose(z, (0, 2, 3, 1)).reshape(N_TOKENS, D)
    idx3, enc, loss11, perp11 = _vq_tc(zf, weight)
    idx = idx3.reshape(N_TOKENS)
    z_q = _sc_gather()(weight, idx)
    z_q_out = jnp.transpose(z_q.reshape(8, 24, 24, D), (0, 3, 1, 2))
    return (z_q_out, loss11[0, 0], (perp11[0, 0], enc, idx))
